# baseline (speedup 1.0000x reference)
import jax
import jax.numpy as jnp
from jax.experimental import pallas as pl
from jax.experimental.pallas import tpu as pltpu

N_BINS = 15
ROWS_PER_BLOCK = 1000


def _ece_kernel(lo_ref, hi_ref, probs_ref, labels_ref, out_ref):
    step = pl.program_id(0)
    nsteps = pl.num_programs(0)

    @pl.when(step == 0)
    def _init():
        out_ref[...] = jnp.zeros_like(out_ref)

    x = probs_ref[...]
    conf = jnp.max(x, axis=1, keepdims=True)
    col = jax.lax.broadcasted_iota(jnp.int32, x.shape, 1)
    pred = jnp.min(jnp.where(x == conf, col, x.shape[1]), axis=1, keepdims=True)
    acc = (pred == labels_ref[...]).astype(jnp.float32)

    lo = lo_ref[...]
    hi = hi_ref[...]
    onehot = ((conf > lo) & (conf <= hi)).astype(jnp.float32)
    num_p = jnp.sum(onehot, axis=0, keepdims=True)
    acc_p = jnp.sum(onehot * acc, axis=0, keepdims=True)
    conf_p = jnp.sum(onehot * conf, axis=0, keepdims=True)

    out_ref[0:1, :] += num_p
    out_ref[1:2, :] += acc_p
    out_ref[2:3, :] += conf_p

    @pl.when(step == nsteps - 1)
    def _finish():
        num = out_ref[0:1, :]
        sacc = out_ref[1:2, :]
        sconf = out_ref[2:3, :]
        safe_n = jnp.maximum(num, 1.0)
        acc_bin = sacc / safe_n
        conf_bin = sconf / safe_n
        has = num > 0.0
        ece = jnp.sum(jnp.where(has, jnp.abs(conf_bin - acc_bin) * num, 0.0))
        out_ref[4:5, :] = jnp.full_like(num, ece)
        out_ref[5:6, :] = jnp.where(has, acc_bin * num, 0.0)
        out_ref[6:7, :] = jnp.where(has, num, 0.0)


def kernel(probs, labels, mode):
    n, c = probs.shape
    r = ROWS_PER_BLOCK
    grid = n // r

    bb = jnp.linspace(0.0, 1.0, N_BINS + 1)
    lo = jnp.full((1, 128), 2.0, dtype=jnp.float32).at[0, :N_BINS].set(bb[:-1])
    hi = jnp.full((1, 128), -1.0, dtype=jnp.float32).at[0, :N_BINS].set(bb[1:])
    labels2 = labels.reshape(n, 1)

    out = pl.pallas_call(
        _ece_kernel,
        grid=(grid,),
        in_specs=[
            pl.BlockSpec((1, 128), lambda i: (0, 0)),
            pl.BlockSpec((1, 128), lambda i: (0, 0)),
            pl.BlockSpec((r, c), lambda i: (i, 0)),
            pl.BlockSpec((r, 1), lambda i: (i, 0)),
        ],
        out_specs=pl.BlockSpec((8, 128), lambda i: (0, 0)),
        out_shape=jax.ShapeDtypeStruct((8, 128), jnp.float32),
        compiler_params=pltpu.CompilerParams(
            dimension_semantics=("arbitrary",),
        ),
    )(lo, hi, probs, labels2)

    ece = out[4, 0:1]
    correct = out[5, 0:N_BINS]
    num = out[6, 0:N_BINS]
    return (ece, correct, num)
